# Initial kernel scaffold; baseline (speedup 1.0000x reference)
import jax
import jax.numpy as jnp
from jax.experimental import pallas as pl


def kernel(x, edge_index, edge_type, batch, basis0, comp0, root0, bias0, basis1, comp1, root1, bias1, basis2, comp2, root2, bias2, basis3, comp3, root3, bias3, W1, b1, W2, b2):
    raise NotImplementedError("write your pallas kernel here")


# speedup vs baseline: 7.3516x; 7.3516x over previous
import functools

import jax
import jax.numpy as jnp
from jax import lax
from jax.experimental import pallas as pl
from jax.experimental.pallas import tpu as pltpu
from jax.experimental.pallas import tpu_sc as plsc

N = 100000
E = 1600000
R = 5
HID = 32
RN = R * N
ZROWS = 6 * N
NC = 2
NS = 16
C = 128
EPT = 102400
E_PAD = EPT * NS
NCHUNK = EPT // C
HEPT = EPT // 2
HALF = N // 2
HALFP = HALF + 16
TRASH = HALF
RPT = 3120
REXTRA = HALF - NS * RPT
CNTP = 512000
CSTRIDE = CNTP // NS

_mesh = plsc.VectorSubcoreMesh(core_axis_name="c", subcore_axis_name="s")
_f32 = jnp.float32
_i32 = jnp.int32
_sc_params = pltpu.CompilerParams(use_tc_tiling_on_sc=False)


def _zero_vmem(ref, n):
    def body(i, _):
        ref[pl.ds(i * 16, 16)] = jnp.zeros((16,), _f32)
        return 0
    lax.fori_loop(0, n // 16, body, 0)


@functools.partial(
    pl.kernel,
    out_type=(
        jax.ShapeDtypeStruct((2 * CNTP,), _f32),
        jax.ShapeDtypeStruct((E_PAD,), _i32),
        jax.ShapeDtypeStruct((E_PAD,), _i32),
    ),
    mesh=_mesh,
    compiler_params=_sc_params,
    scratch_types=[
        pltpu.VMEM((C,), _i32),
        pltpu.VMEM((C,), _i32),
        pltpu.VMEM((C,), _i32),
        pltpu.VMEM((C,), _i32),
        pltpu.VMEM((C,), _i32),
        pltpu.VMEM((C,), _f32),
        pltpu.VMEM((CSTRIDE,), _f32),
        pltpu.VMEM_SHARED((CNTP,), _f32),
    ],
)
def _p1_counts(src_h, dst_h, et_h, cnt_out, gidx_out, cidx_out,
               sv, dv, tv, gv, cv, ones, zbuf, cnt_sh):
    c = lax.axis_index("c")
    s = lax.axis_index("s")
    _zero_vmem(zbuf, CSTRIDE)
    pltpu.sync_copy(zbuf, cnt_sh.at[pl.ds(s * CSTRIDE, CSTRIDE)])
    for k in range(C // 16):
        ones[pl.ds(k * 16, 16)] = jnp.ones((16,), _f32)
    plsc.subcore_barrier()

    def chunk(i, _):
        base = s * EPT + c * HEPT + i * C
        pltpu.sync_copy(src_h.at[pl.ds(base, C)], sv)
        pltpu.sync_copy(dst_h.at[pl.ds(base, C)], dv)
        pltpu.sync_copy(et_h.at[pl.ds(base, C)], tv)
        for k in range(C // 16):
            sl = pl.ds(k * 16, 16)
            t = tv[sl]
            d = dv[sl]
            gv[sl] = t * N + sv[sl]
            cv[sl] = jnp.where(d < N, t * N + d, RN)
        pltpu.sync_copy(gv, gidx_out.at[pl.ds(base, C)])
        pltpu.sync_copy(cv, cidx_out.at[pl.ds(base, C)])
        pltpu.sync_copy(ones, cnt_sh.at[cv], add=True)
        return 0

    lax.fori_loop(0, NCHUNK // 2, chunk, 0)
    plsc.subcore_barrier()
    pltpu.sync_copy(cnt_sh.at[pl.ds(s * CSTRIDE, CSTRIDE)],
                    cnt_out.at[pl.ds(c * CNTP + s * CSTRIDE, CSTRIDE)])


def _p2_body(c_ref, inv_ref):
    cnt = c_ref[0] + c_ref[1]
    inv_ref[...] = 1.0 / jnp.maximum(cnt, 1.0)


_p2_inv = pl.pallas_call(
    _p2_body,
    grid=(CNTP // 128 // 8,),
    in_specs=[pl.BlockSpec((2, 8, 128), lambda i: (0, i, 0))],
    out_specs=pl.BlockSpec((8, 128), lambda i: (i, 0)),
    out_shape=jax.ShapeDtypeStruct((CNTP // 128, 128), _f32),
)


@functools.partial(
    pl.kernel,
    out_type=jax.ShapeDtypeStruct((E_PAD,), _f32),
    mesh=_mesh,
    compiler_params=_sc_params,
    scratch_types=[
        pltpu.VMEM((C,), _i32),
        pltpu.VMEM((C,), _f32),
        pltpu.SemaphoreType.DMA,
    ],
)
def _p3_scale(cidx_h, inv_h, scale_out, cv, sv, sem):
    c = lax.axis_index("c")
    s = lax.axis_index("s")

    def chunk(i, _):
        base = s * EPT + c * HEPT + i * C
        pltpu.sync_copy(cidx_h.at[pl.ds(base, C)], cv)
        pltpu.async_copy(inv_h.at[cv], sv, sem).wait()
        pltpu.sync_copy(sv, scale_out.at[pl.ds(base, C)])
        return 0

    lax.fori_loop(0, NCHUNK // 2, chunk, 0)


def _make_proj(nin, apply_tanh):
    def body(h_ref, basis_ref, comp_ref, b_ref, z_ref):
        cm = comp_ref[0, 0]
        w = (cm[0] * basis_ref[0] + cm[1] * basis_ref[1]
             + cm[2] * basis_ref[2])
        h = h_ref[...]
        if apply_tanh:
            h = jnp.tanh(h)
        z_ref[...] = jnp.dot(h, w, preferred_element_type=_f32) + b_ref[0, 0]

    blk = 1000
    nb = N // blk
    return pl.pallas_call(
        body,
        grid=(6, nb),
        in_specs=[
            pl.BlockSpec((blk, nin), lambda i, j: (j, 0)),
            pl.BlockSpec((3, nin, HID), lambda i, j: (0, 0, 0)),
            pl.BlockSpec((1, 1, 3), lambda i, j: (i, 0, 0)),
            pl.BlockSpec((1, 1, HID), lambda i, j: (i, 0, 0)),
        ],
        out_specs=pl.BlockSpec((blk, HID), lambda i, j: (i * nb + j, 0)),
        out_shape=jax.ShapeDtypeStruct((ZROWS, HID), _f32),
    )


_proj_first = _make_proj(4, False)
_proj_rest = _make_proj(HID, True)


@functools.partial(
    pl.kernel,
    out_type=jax.ShapeDtypeStruct((N, HID), _f32),
    mesh=_mesh,
    compiler_params=_sc_params,
    scratch_types=[
        pltpu.VMEM((C,), _i32),
        pltpu.VMEM((C,), _i32),
        pltpu.VMEM((C,), _i32),
        pltpu.VMEM((C,), _f32),
        pltpu.VMEM((C, HID), _f32),
        pltpu.VMEM((16, HID), _f32),
        pltpu.VMEM_SHARED((HALFP, HID), _f32),
        pltpu.SemaphoreType.DMA,
    ],
)
def _agg(z_h, gidx_h, dst_h, scale_h, out_h,
         gv, dv, dlv, sv, rows, ztr, acc, sem):
    c = lax.axis_index("c")
    s = lax.axis_index("s")
    lo = c * HALF
    pltpu.sync_copy(z_h.at[pl.ds(5 * N + lo + s * RPT, RPT)],
                    acc.at[pl.ds(s * RPT, RPT)])

    @pl.when(s == 0)
    def _():
        pltpu.sync_copy(z_h.at[pl.ds(5 * N + lo + NS * RPT, REXTRA)],
                        acc.at[pl.ds(NS * RPT, REXTRA)])
        for k in range(16):
            ztr[k, pl.ds(0, 16)] = jnp.zeros((16,), _f32)
            ztr[k, pl.ds(16, 16)] = jnp.zeros((16,), _f32)
        pltpu.sync_copy(ztr, acc.at[pl.ds(HALF, 16)])

    plsc.subcore_barrier()

    def chunk(i, _):
        base = s * EPT + i * C
        pltpu.sync_copy(gidx_h.at[pl.ds(base, C)], gv)
        pltpu.sync_copy(dst_h.at[pl.ds(base, C)], dv)
        pltpu.sync_copy(scale_h.at[pl.ds(base, C)], sv)
        pltpu.async_copy(z_h.at[gv], rows, sem).wait()
        for k in range(C // 16):
            sl = pl.ds(k * 16, 16)
            d = dv[sl]
            inb = jnp.logical_and(d >= lo, d < lo + HALF)
            dlv[sl] = jnp.where(inb, d - lo, TRASH)

        def scale_grp(k, _):
            b16 = k * 16
            svec = sv[pl.ds(b16, 16)]
            for j in range(16):
                sc = svec[j]
                e = b16 + j
                rows[e, pl.ds(0, 16)] = rows[e, pl.ds(0, 16)] * sc
                rows[e, pl.ds(16, 16)] = rows[e, pl.ds(16, 16)] * sc
            return 0

        lax.fori_loop(0, C // 16, scale_grp, 0)
        pltpu.sync_copy(rows, acc.at[dlv], add=True)
        return 0

    lax.fori_loop(0, NCHUNK, chunk, 0)
    plsc.subcore_barrier()
    pltpu.sync_copy(acc.at[pl.ds(s * RPT, RPT)],
                    out_h.at[pl.ds(lo + s * RPT, RPT)])

    @pl.when(s == 0)
    def _():
        pltpu.sync_copy(acc.at[pl.ds(NS * RPT, REXTRA)],
                        out_h.at[pl.ds(lo + NS * RPT, REXTRA)])


def _mlp_body(u0, u1, u2, u3, i0, i1, i2, i3, w1_ref, b1_ref, w2_ref, b2_ref,
              out_ref):
    feats = [u0, u1, u2, u3, i0, i1, i2, i3]
    acc = jnp.broadcast_to(b1_ref[...], (1000, 128))
    for k, f in enumerate(feats):
        h = jnp.tanh(f[...])
        acc = acc + jnp.dot(h, w1_ref[pl.ds(32 * k, 32), :],
                            preferred_element_type=_f32)
    r = jnp.maximum(acc, 0.0)
    o = jnp.sum(r * w2_ref[...], axis=1, keepdims=True) + b2_ref[0, 0]
    out_ref[...] = o


def _mlp(us, its, w1, b1, w2t, b2):
    nq = N // 4
    specs = [pl.BlockSpec((1000, HID), lambda i: (i, 0))] * 8
    specs += [
        pl.BlockSpec((256, 128), lambda i: (0, 0)),
        pl.BlockSpec((1, 128), lambda i: (0, 0)),
        pl.BlockSpec((1, 128), lambda i: (0, 0)),
        pl.BlockSpec((1, 1), lambda i: (0, 0)),
    ]
    return pl.pallas_call(
        _mlp_body,
        grid=(nq // 1000,),
        in_specs=specs,
        out_specs=pl.BlockSpec((1000, 1), lambda i: (i, 0)),
        out_shape=jax.ShapeDtypeStruct((nq, 1), _f32),
    )(*us, *its, w1, b1, w2t, b2)


def kernel(x, edge_index, edge_type, batch,
           basis0, comp0, root0, bias0, basis1, comp1, root1, bias1,
           basis2, comp2, root2, bias2, basis3, comp3, root3, bias3,
           W1, b1, W2, b2):
    src = edge_index[0]
    dst = edge_index[1]
    npad = E_PAD - E
    src_p = jnp.concatenate([src, jnp.zeros((npad,), _i32)])
    dst_p = jnp.concatenate([dst, jnp.full((npad,), N, _i32)])
    et_p = jnp.concatenate([edge_type, jnp.zeros((npad,), _i32)])

    cnt_part, gidx, cidx = _p1_counts(src_p, dst_p, et_p)
    inv = _p2_inv(cnt_part.reshape(2, CNTP // 128, 128)).reshape(CNTP)
    scale = _p3_scale(cidx, inv)

    def wprep(basis, comp, root, bias):
        b6 = jnp.concatenate([basis, root[None]], axis=0)
        c6 = jnp.concatenate([
            jnp.concatenate([comp, jnp.zeros((R, 1), _f32)], axis=1),
            jnp.array([[0.0, 0.0, 1.0]], _f32)], axis=0).reshape(6, 1, 3)
        bb = jnp.concatenate([jnp.zeros((R, HID), _f32), bias[None]],
                             axis=0).reshape(6, 1, HID)
        return b6, c6, bb

    layers = [(basis0, comp0, root0, bias0), (basis1, comp1, root1, bias1),
              (basis2, comp2, root2, bias2), (basis3, comp3, root3, bias3)]
    h = x
    pre = []
    for li, (bs, cp, rt, bi) in enumerate(layers):
        b6, c6, bb = wprep(bs, cp, rt, bi)
        proj = _proj_first if li == 0 else _proj_rest
        z = proj(h, b6, c6, bb)
        h = _agg(z, gidx, dst_p, scale)
        pre.append(h)

    nq = N // 4
    us = [p.reshape(nq, 4, HID)[:, 0, :] for p in pre]
    its = [p.reshape(nq, 4, HID)[:, 1, :] for p in pre]
    return _mlp(us, its, W1, b1.reshape(1, 128), W2.reshape(1, 128),
                b2.reshape(1, 1))


# speedup vs baseline: 10.3490x; 1.4077x over previous
import functools

import jax
import jax.numpy as jnp
from jax import lax
from jax.experimental import pallas as pl
from jax.experimental.pallas import tpu as pltpu
from jax.experimental.pallas import tpu_sc as plsc

N = 100000
E = 1600000
R = 5
HID = 32
RN = R * N
ZROWS = 6 * N
NC = 2
NS = 16
C = 128
EPT = 102400
E_PAD = EPT * NS
NCHUNK = EPT // C
HEPT = EPT // 2
HALF = N // 2
HALFP = HALF + 16
TRASH = HALF
RPT = 3120
REXTRA = HALF - NS * RPT
CNTP = 512000
CSTRIDE = CNTP // NS

_mesh = plsc.VectorSubcoreMesh(core_axis_name="c", subcore_axis_name="s")
_f32 = jnp.float32
_i32 = jnp.int32
_sc_params = pltpu.CompilerParams(use_tc_tiling_on_sc=False)


def _zero_vmem(ref, n):
    def body(i, _):
        ref[pl.ds(i * 16, 16)] = jnp.zeros((16,), _f32)
        return 0
    lax.fori_loop(0, n // 16, body, 0)


@functools.partial(
    pl.kernel,
    out_type=(
        jax.ShapeDtypeStruct((2 * CNTP,), _f32),
        jax.ShapeDtypeStruct((E_PAD,), _i32),
        jax.ShapeDtypeStruct((E_PAD,), _i32),
    ),
    mesh=_mesh,
    compiler_params=_sc_params,
    scratch_types=[
        pltpu.VMEM((C,), _i32),
        pltpu.VMEM((C,), _i32),
        pltpu.VMEM((C,), _i32),
        pltpu.VMEM((C,), _i32),
        pltpu.VMEM((C,), _i32),
        pltpu.VMEM((C,), _f32),
        pltpu.VMEM((CSTRIDE,), _f32),
        pltpu.VMEM_SHARED((CNTP,), _f32),
    ],
)
def _p1_counts(src_h, dst_h, et_h, cnt_out, gidx_out, cidx_out,
               sv, dv, tv, gv, cv, ones, zbuf, cnt_sh):
    c = lax.axis_index("c")
    s = lax.axis_index("s")
    _zero_vmem(zbuf, CSTRIDE)
    pltpu.sync_copy(zbuf, cnt_sh.at[pl.ds(s * CSTRIDE, CSTRIDE)])
    for k in range(C // 16):
        ones[pl.ds(k * 16, 16)] = jnp.ones((16,), _f32)
    plsc.subcore_barrier()

    def chunk(i, _):
        base = s * EPT + c * HEPT + i * C
        pltpu.sync_copy(src_h.at[pl.ds(base, C)], sv)
        pltpu.sync_copy(dst_h.at[pl.ds(base, C)], dv)
        pltpu.sync_copy(et_h.at[pl.ds(base, C)], tv)
        for k in range(C // 16):
            sl = pl.ds(k * 16, 16)
            t = tv[sl]
            d = dv[sl]
            gv[sl] = t * N + sv[sl]
            cv[sl] = jnp.where(d < N, t * N + d, RN)
        pltpu.sync_copy(gv, gidx_out.at[pl.ds(base, C)])
        pltpu.sync_copy(cv, cidx_out.at[pl.ds(base, C)])
        pltpu.sync_copy(ones, cnt_sh.at[cv], add=True)
        return 0

    lax.fori_loop(0, NCHUNK // 2, chunk, 0)
    plsc.subcore_barrier()
    pltpu.sync_copy(cnt_sh.at[pl.ds(s * CSTRIDE, CSTRIDE)],
                    cnt_out.at[pl.ds(c * CNTP + s * CSTRIDE, CSTRIDE)])


def _p2_body(c_ref, inv_ref):
    cnt = c_ref[0] + c_ref[1]
    inv_ref[...] = 1.0 / jnp.maximum(cnt, 1.0)


_p2_inv = pl.pallas_call(
    _p2_body,
    grid=(CNTP // 128 // 8,),
    in_specs=[pl.BlockSpec((2, 8, 128), lambda i: (0, i, 0))],
    out_specs=pl.BlockSpec((8, 128), lambda i: (i, 0)),
    out_shape=jax.ShapeDtypeStruct((CNTP // 128, 128), _f32),
)


@functools.partial(
    pl.kernel,
    out_type=jax.ShapeDtypeStruct((E_PAD,), _f32),
    mesh=_mesh,
    compiler_params=_sc_params,
    scratch_types=[
        pltpu.VMEM((C,), _i32),
        pltpu.VMEM((C,), _f32),
        pltpu.SemaphoreType.DMA,
    ],
)
def _p3_scale(cidx_h, inv_h, scale_out, cv, sv, sem):
    c = lax.axis_index("c")
    s = lax.axis_index("s")

    def chunk(i, _):
        base = s * EPT + c * HEPT + i * C
        pltpu.sync_copy(cidx_h.at[pl.ds(base, C)], cv)
        pltpu.async_copy(inv_h.at[cv], sv, sem).wait()
        pltpu.sync_copy(sv, scale_out.at[pl.ds(base, C)])
        return 0

    lax.fori_loop(0, NCHUNK // 2, chunk, 0)


def _make_proj(nin, apply_tanh):
    def body(h_ref, basis_ref, comp_ref, b_ref, z_ref):
        cm = comp_ref[0, 0]
        w = (cm[0] * basis_ref[0] + cm[1] * basis_ref[1]
             + cm[2] * basis_ref[2])
        h = h_ref[...]
        if apply_tanh:
            h = jnp.tanh(h)
        z_ref[...] = jnp.dot(h, w, preferred_element_type=_f32) + b_ref[0, 0]

    blk = 1000
    nb = N // blk
    return pl.pallas_call(
        body,
        grid=(6, nb),
        in_specs=[
            pl.BlockSpec((blk, nin), lambda i, j: (j, 0)),
            pl.BlockSpec((3, nin, HID), lambda i, j: (0, 0, 0)),
            pl.BlockSpec((1, 1, 3), lambda i, j: (i, 0, 0)),
            pl.BlockSpec((1, 1, HID), lambda i, j: (i, 0, 0)),
        ],
        out_specs=pl.BlockSpec((blk, HID), lambda i, j: (i * nb + j, 0)),
        out_shape=jax.ShapeDtypeStruct((ZROWS, HID), _f32),
    )


_proj_first = _make_proj(4, False)
_proj_rest = _make_proj(HID, True)


@functools.partial(
    pl.kernel,
    out_type=jax.ShapeDtypeStruct((N, HID), _f32),
    mesh=_mesh,
    compiler_params=_sc_params,
    scratch_types=[
        [pltpu.VMEM((C,), _i32)] * 2,
        [pltpu.VMEM((C,), _i32)] * 2,
        [pltpu.VMEM((C,), _i32)] * 2,
        [pltpu.VMEM((C,), _f32)] * 2,
        [pltpu.VMEM((C, HID), _f32)] * 2,
        pltpu.VMEM((16, HID), _f32),
        pltpu.VMEM_SHARED((HALFP, HID), _f32),
        [pltpu.SemaphoreType.DMA] * 2,
        [pltpu.SemaphoreType.DMA] * 2,
    ],
)
def _agg(z_h, gidx_h, dst_h, scale_h, out_h,
         gv, dv, dlv, sv, rows, ztr, acc, sem_r, sem_g):
    c = lax.axis_index("c")
    s = lax.axis_index("s")
    lo = c * HALF
    pltpu.sync_copy(z_h.at[pl.ds(5 * N + lo + s * RPT, RPT)],
                    acc.at[pl.ds(s * RPT, RPT)])

    @pl.when(s == 0)
    def _():
        pltpu.sync_copy(z_h.at[pl.ds(5 * N + lo + NS * RPT, REXTRA)],
                        acc.at[pl.ds(NS * RPT, REXTRA)])
        for k in range(16):
            ztr[k, pl.ds(0, 16)] = jnp.zeros((16,), _f32)
            ztr[k, pl.ds(16, 16)] = jnp.zeros((16,), _f32)
        pltpu.sync_copy(ztr, acc.at[pl.ds(HALF, 16)])

    plsc.subcore_barrier()

    def rec_load(i, b):
        base = s * EPT + i * C
        pltpu.async_copy(gidx_h.at[pl.ds(base, C)], gv[b], sem_r[b])
        pltpu.async_copy(dst_h.at[pl.ds(base, C)], dv[b], sem_r[b])
        pltpu.async_copy(scale_h.at[pl.ds(base, C)], sv[b], sem_r[b])

    def rec_wait(i, b):
        base = s * EPT + i * C
        pltpu.make_async_copy(gidx_h.at[pl.ds(base, C)], gv[b], sem_r[b]).wait()
        pltpu.make_async_copy(dst_h.at[pl.ds(base, C)], dv[b], sem_r[b]).wait()
        pltpu.make_async_copy(scale_h.at[pl.ds(base, C)], sv[b],
                              sem_r[b]).wait()

    rec_load(0, 0)
    rec_load(1, 1)
    rec_wait(0, 0)
    pltpu.async_copy(z_h.at[gv[0]], rows[0], sem_g[0])

    def pair(i2, _):
        for p in (0, 1):
            i = i2 * 2 + p
            q = 1 - p
            pltpu.make_async_copy(z_h.at[gv[p]], rows[p], sem_g[p]).wait()

            @pl.when(i < NCHUNK - 1)
            def _():
                rec_wait(i + 1, q)
                pltpu.async_copy(z_h.at[gv[q]], rows[q], sem_g[q])

            for k in range(C // 16):
                sl = pl.ds(k * 16, 16)
                d = dv[p][sl]
                inb = jnp.logical_and(d >= lo, d < lo + HALF)
                dlv[p][sl] = jnp.where(inb, d - lo, TRASH)

            def scale_grp(k, _):
                b16 = k * 16
                svec = sv[p][pl.ds(b16, 16)]
                for j in range(16):
                    sc = svec[j]
                    e = b16 + j
                    rows[p][e, pl.ds(0, 16)] = rows[p][e, pl.ds(0, 16)] * sc
                    rows[p][e, pl.ds(16, 16)] = rows[p][e, pl.ds(16, 16)] * sc
                return 0

            lax.fori_loop(0, C // 16, scale_grp, 0)
            pltpu.sync_copy(rows[p], acc.at[dlv[p]], add=True)

            @pl.when(i < NCHUNK - 2)
            def _():
                rec_load(i + 2, p)

        return 0

    lax.fori_loop(0, NCHUNK // 2, pair, 0)
    plsc.subcore_barrier()
    pltpu.sync_copy(acc.at[pl.ds(s * RPT, RPT)],
                    out_h.at[pl.ds(lo + s * RPT, RPT)])

    @pl.when(s == 0)
    def _():
        pltpu.sync_copy(acc.at[pl.ds(NS * RPT, REXTRA)],
                        out_h.at[pl.ds(lo + NS * RPT, REXTRA)])


def _mlp_body(u0, u1, u2, u3, i0, i1, i2, i3, w1_ref, b1_ref, w2_ref, b2_ref,
              out_ref):
    feats = [u0, u1, u2, u3, i0, i1, i2, i3]
    acc = jnp.broadcast_to(b1_ref[...], (1000, 128))
    for k, f in enumerate(feats):
        h = jnp.tanh(f[...])
        acc = acc + jnp.dot(h, w1_ref[pl.ds(32 * k, 32), :],
                            preferred_element_type=_f32)
    r = jnp.maximum(acc, 0.0)
    o = jnp.sum(r * w2_ref[...], axis=1, keepdims=True) + b2_ref[0, 0]
    out_ref[...] = o


def _mlp(us, its, w1, b1, w2t, b2):
    nq = N // 4
    specs = [pl.BlockSpec((1000, HID), lambda i: (i, 0))] * 8
    specs += [
        pl.BlockSpec((256, 128), lambda i: (0, 0)),
        pl.BlockSpec((1, 128), lambda i: (0, 0)),
        pl.BlockSpec((1, 128), lambda i: (0, 0)),
        pl.BlockSpec((1, 1), lambda i: (0, 0)),
    ]
    return pl.pallas_call(
        _mlp_body,
        grid=(nq // 1000,),
        in_specs=specs,
        out_specs=pl.BlockSpec((1000, 1), lambda i: (i, 0)),
        out_shape=jax.ShapeDtypeStruct((nq, 1), _f32),
    )(*us, *its, w1, b1, w2t, b2)


def kernel(x, edge_index, edge_type, batch,
           basis0, comp0, root0, bias0, basis1, comp1, root1, bias1,
           basis2, comp2, root2, bias2, basis3, comp3, root3, bias3,
           W1, b1, W2, b2):
    src = edge_index[0]
    dst = edge_index[1]
    npad = E_PAD - E
    src_p = jnp.concatenate([src, jnp.zeros((npad,), _i32)])
    dst_p = jnp.concatenate([dst, jnp.full((npad,), N, _i32)])
    et_p = jnp.concatenate([edge_type, jnp.zeros((npad,), _i32)])

    cnt_part, gidx, cidx = _p1_counts(src_p, dst_p, et_p)
    inv = _p2_inv(cnt_part.reshape(2, CNTP // 128, 128)).reshape(CNTP)
    scale = _p3_scale(cidx, inv)

    def wprep(basis, comp, root, bias):
        b6 = jnp.concatenate([basis, root[None]], axis=0)
        c6 = jnp.concatenate([
            jnp.concatenate([comp, jnp.zeros((R, 1), _f32)], axis=1),
            jnp.array([[0.0, 0.0, 1.0]], _f32)], axis=0).reshape(6, 1, 3)
        bb = jnp.concatenate([jnp.zeros((R, HID), _f32), bias[None]],
                             axis=0).reshape(6, 1, HID)
        return b6, c6, bb

    layers = [(basis0, comp0, root0, bias0), (basis1, comp1, root1, bias1),
              (basis2, comp2, root2, bias2), (basis3, comp3, root3, bias3)]
    h = x
    pre = []
    for li, (bs, cp, rt, bi) in enumerate(layers):
        b6, c6, bb = wprep(bs, cp, rt, bi)
        proj = _proj_first if li == 0 else _proj_rest
        z = proj(h, b6, c6, bb)
        h = _agg(z, gidx, dst_p, scale)
        pre.append(h)

    nq = N // 4
    us = [p.reshape(nq, 4, HID)[:, 0, :] for p in pre]
    its = [p.reshape(nq, 4, HID)[:, 1, :] for p in pre]
    return _mlp(us, its, W1, b1.reshape(1, 128), W2.reshape(1, 128),
                b2.reshape(1, 1))
